# baseline (speedup 1.0000x reference)
import functools

import numpy as np
import jax
import jax.numpy as jnp
from jax.experimental import pallas as pl
from jax.experimental.pallas import tpu as pltpu

_N_CTRL = 7
_CTRL_X = np.linspace(-1.0, 1.0, _N_CTRL).astype(np.float32)
_DX = (_CTRL_X[1:] - _CTRL_X[:-1]).astype(np.float32)

_INV_12_92 = 1.0 / 12.92
_INV_1_055 = 1.0 / 1.055
_THIRD = 1.0 / 3.0

_WB = 128


def _curve_tables(w):
    ctrl_x = jnp.linspace(-1.0, 1.0, _N_CTRL)
    strengths = (0.25, 0.15, 0.15)
    curve_key = jax.random.key(1)
    rows = []
    for ch in range(3):
        ck = jax.random.fold_in(curve_key, ch)
        noise = jax.random.normal(ck, (_N_CTRL - 2,), dtype=jnp.float32) * strengths[ch]
        ctrl_y = ctrl_x.at[1:-1].add(noise)
        ctrl_y = jnp.sort(ctrl_y)
        ctrl_y = jnp.clip(ctrl_y, -1.0, 1.0)
        ctrl_y = ctrl_y.at[0].set(-1.0).at[-1].set(1.0)
        slope = (ctrl_y[1:] - ctrl_y[:-1]) / (ctrl_x[1:] - ctrl_x[:-1] + 1e-8)
        rows.append(jnp.pad(slope * _THIRD, (0, 2)))
        rows.append(jnp.pad(ctrl_y[:-1], (0, 2)))
    return jnp.broadcast_to(jnp.stack(rows)[:, :, None], (6, 8, w))


def _srgb_to_linear(c):
    u = (c + 0.055) * _INV_1_055
    return jnp.where(c <= 0.04045, c * _INV_12_92, jnp.exp(2.4 * jnp.log(u)))


def _cbrt_pos(x):
    return jnp.exp(jnp.log(x) * _THIRD)


def _apply_pwl(v, tbl_ref, ch):
    t = (v + 1.0) * 3.0
    tf = jnp.minimum(jnp.floor(t), float(_N_CTRL - 2))
    idx = tf.astype(jnp.int32)
    s = jnp.take_along_axis(tbl_ref[2 * ch], idx, axis=0)
    y = jnp.take_along_axis(tbl_ref[2 * ch + 1], idx, axis=0)
    return y + (t - tf) * s


def _flip_lanes(v):
    n = _WB
    h, w = v.shape
    rev = (n - 1) - jax.lax.broadcasted_iota(jnp.int32, (h, n), 1)
    tiles = []
    for t in range(w // n - 1, -1, -1):
        tiles.append(jnp.take_along_axis(v[:, t * n:(t + 1) * n], rev, axis=1))
    return jnp.concatenate(tiles, axis=-1)


def _body(tbl_ref, x_ref, out_in_ref, out_tg_ref, *, flip_w):
  for bi in range(x_ref.shape[0]):
    r = x_ref[bi, 0]
    g = x_ref[bi, 1]
    b = x_ref[bi, 2]
    if flip_w:
        r = _flip_lanes(r)
        g = _flip_lanes(g)
        b = _flip_lanes(b)

    rl = _srgb_to_linear(r)
    gl = _srgb_to_linear(g)
    bl = _srgb_to_linear(b)

    l = 0.4122214708 * rl + 0.5363325363 * gl + 0.0514459929 * bl
    m = 0.2119034982 * rl + 0.6806995451 * gl + 0.1073969566 * bl
    s = 0.0883024619 * rl + 0.2817188376 * gl + 0.6299787005 * bl
    l_ = _cbrt_pos(l)
    m_ = _cbrt_pos(m)
    s_ = _cbrt_pos(s)

    L2 = (2.0 * 0.2104542553) * l_ + (2.0 * 0.7936177850) * m_ - (2.0 * 0.0040720468) * s_
    a2 = (2.5 * 1.9779984951) * l_ - (2.5 * 2.4285922050) * m_ + (2.5 * 0.4505937099) * s_
    b2 = (2.5 * 0.0259040371) * l_ + (2.5 * 0.7827717662) * m_ - (2.5 * 0.8086757660) * s_

    Lp = jnp.clip(L2 - 1.0, -1.0, 1.0)
    ap = jnp.clip(a2, -1.0, 1.0)
    bp = jnp.clip(b2, -1.0, 1.0)

    out_tg_ref[bi, 0] = Lp
    out_tg_ref[bi, 1] = ap
    out_tg_ref[bi, 2] = bp

    out_in_ref[bi, 0] = _apply_pwl(Lp, tbl_ref, 0)
    out_in_ref[bi, 1] = _apply_pwl(ap, tbl_ref, 1)
    out_in_ref[bi, 2] = _apply_pwl(bp, tbl_ref, 2)


@jax.jit
def kernel(x):
    B, C, H, W = x.shape

    rng = np.random.RandomState(42)
    flips = rng.rand(3) < 0.5
    if flips[1]:
        x = jnp.flip(x, axis=2)
    if flips[2]:
        x = jnp.rot90(x, 1, axes=(2, 3))
        B, C, H, W = x.shape

    tables = _curve_tables(W)

    bb = 2
    grid = (B // bb,)
    spec = pl.BlockSpec((bb, C, H, W), lambda i: (i, 0, 0, 0))
    out = pl.pallas_call(
        functools.partial(_body, flip_w=bool(flips[0])),
        grid=grid,
        in_specs=[
            pl.BlockSpec((6, 8, W), lambda i: (0, 0, 0)),
            spec,
        ],
        out_specs=[spec, spec],
        out_shape=[
            jax.ShapeDtypeStruct((B, C, H, W), x.dtype),
            jax.ShapeDtypeStruct((B, C, H, W), x.dtype),
        ],
        compiler_params=pltpu.CompilerParams(
            dimension_semantics=("parallel",),
        ),
    )(tables, x)
    return (out[0], out[1])


# speedup vs baseline: 1.0067x; 1.0067x over previous
import functools

import numpy as np
import jax
import jax.numpy as jnp
from jax.experimental import pallas as pl
from jax.experimental.pallas import tpu as pltpu

_N_CTRL = 7
_CTRL_X = np.linspace(-1.0, 1.0, _N_CTRL).astype(np.float32)
_DX = (_CTRL_X[1:] - _CTRL_X[:-1]).astype(np.float32)

_INV_12_92 = 1.0 / 12.92
_INV_1_055 = 1.0 / 1.055
_THIRD = 1.0 / 3.0

_WB = 128


def _curve_tables(w):
    ctrl_x = jnp.linspace(-1.0, 1.0, _N_CTRL)
    strengths = (0.25, 0.15, 0.15)
    curve_key = jax.random.key(1)
    rows = []
    for ch in range(3):
        ck = jax.random.fold_in(curve_key, ch)
        noise = jax.random.normal(ck, (_N_CTRL - 2,), dtype=jnp.float32) * strengths[ch]
        ctrl_y = ctrl_x.at[1:-1].add(noise)
        ctrl_y = jnp.sort(ctrl_y)
        ctrl_y = jnp.clip(ctrl_y, -1.0, 1.0)
        ctrl_y = ctrl_y.at[0].set(-1.0).at[-1].set(1.0)
        slope = (ctrl_y[1:] - ctrl_y[:-1]) / (ctrl_x[1:] - ctrl_x[:-1] + 1e-8)
        rows.append(jnp.pad(slope * _THIRD, (0, 2)))
        rows.append(jnp.pad(ctrl_y[:-1], (0, 2)))
    return jnp.broadcast_to(jnp.stack(rows)[:, :, None], (6, 8, w))


def _srgb_to_linear(c):
    u = (c + 0.055) * _INV_1_055
    return jnp.where(c <= 0.04045, c * _INV_12_92, jnp.exp(2.4 * jnp.log(u)))


def _cbrt_pos(x):
    return jnp.exp(jnp.log(x) * _THIRD)


def _apply_pwl(v, tbl_ref, ch):
    t = (v + 1.0) * 3.0
    tf = jnp.minimum(jnp.floor(t), float(_N_CTRL - 2))
    idx = tf.astype(jnp.int32)
    s = jnp.take_along_axis(tbl_ref[2 * ch], idx, axis=0)
    y = jnp.take_along_axis(tbl_ref[2 * ch + 1], idx, axis=0)
    return y + (t - tf) * s


def _flip_lanes(v):
    n = _WB
    h, w = v.shape
    rev = (n - 1) - jax.lax.broadcasted_iota(jnp.int32, (h, n), 1)
    tiles = []
    for t in range(w // n - 1, -1, -1):
        tiles.append(jnp.take_along_axis(v[:, t * n:(t + 1) * n], rev, axis=1))
    return jnp.concatenate(tiles, axis=-1)


def _body(tbl_ref, x_ref, out_in_ref, out_tg_ref, *, flip_w):
  for bi in range(x_ref.shape[0]):
    r = x_ref[bi, 0]
    g = x_ref[bi, 1]
    b = x_ref[bi, 2]
    if flip_w:
        r = _flip_lanes(r)
        g = _flip_lanes(g)
        b = _flip_lanes(b)

    rl = _srgb_to_linear(r)
    gl = _srgb_to_linear(g)
    bl = _srgb_to_linear(b)

    l = 0.4122214708 * rl + 0.5363325363 * gl + 0.0514459929 * bl
    m = 0.2119034982 * rl + 0.6806995451 * gl + 0.1073969566 * bl
    s = 0.0883024619 * rl + 0.2817188376 * gl + 0.6299787005 * bl
    l_ = _cbrt_pos(l)
    m_ = _cbrt_pos(m)
    s_ = _cbrt_pos(s)

    L2 = (2.0 * 0.2104542553) * l_ + (2.0 * 0.7936177850) * m_ - (2.0 * 0.0040720468) * s_
    a2 = (2.5 * 1.9779984951) * l_ - (2.5 * 2.4285922050) * m_ + (2.5 * 0.4505937099) * s_
    b2 = (2.5 * 0.0259040371) * l_ + (2.5 * 0.7827717662) * m_ - (2.5 * 0.8086757660) * s_

    Lp = jnp.clip(L2 - 1.0, -1.0, 1.0)
    ap = jnp.clip(a2, -1.0, 1.0)
    bp = jnp.clip(b2, -1.0, 1.0)

    out_tg_ref[bi, 0] = Lp
    out_tg_ref[bi, 1] = ap
    out_tg_ref[bi, 2] = bp

    out_in_ref[bi, 0] = _apply_pwl(Lp, tbl_ref, 0)
    out_in_ref[bi, 1] = _apply_pwl(ap, tbl_ref, 1)
    out_in_ref[bi, 2] = _apply_pwl(bp, tbl_ref, 2)


@jax.jit
def kernel(x):
    B, C, H, W = x.shape

    rng = np.random.RandomState(42)
    flips = rng.rand(3) < 0.5
    if flips[1]:
        x = jnp.flip(x, axis=2)
    if flips[2]:
        x = jnp.rot90(x, 1, axes=(2, 3))
        B, C, H, W = x.shape

    tables = _curve_tables(W)

    hb = 512
    grid = (B, H // hb)
    spec = pl.BlockSpec((1, C, hb, W), lambda i, j: (i, 0, j, 0))
    out = pl.pallas_call(
        functools.partial(_body, flip_w=bool(flips[0])),
        grid=grid,
        in_specs=[
            pl.BlockSpec((6, 8, W), lambda i, j: (0, 0, 0)),
            spec,
        ],
        out_specs=[spec, spec],
        out_shape=[
            jax.ShapeDtypeStruct((B, C, H, W), x.dtype),
            jax.ShapeDtypeStruct((B, C, H, W), x.dtype),
        ],
        compiler_params=pltpu.CompilerParams(
            dimension_semantics=("parallel", "parallel"),
        ),
    )(tables, x)
    return (out[0], out[1])


# speedup vs baseline: 1.2160x; 1.2079x over previous
import functools

import numpy as np
import jax
import jax.numpy as jnp
from jax.experimental import pallas as pl
from jax.experimental.pallas import tpu as pltpu

_N_CTRL = 7
_CTRL_X = np.linspace(-1.0, 1.0, _N_CTRL).astype(np.float32)
_DX = (_CTRL_X[1:] - _CTRL_X[:-1]).astype(np.float32)

_INV_12_92 = 1.0 / 12.92
_INV_1_055 = 1.0 / 1.055
_THIRD = 1.0 / 3.0

_WB = 128


def _curve_tables(w):
    with jax.ensure_compile_time_eval():
        ctrl_x = jnp.linspace(-1.0, 1.0, _N_CTRL)
        strengths = (0.25, 0.15, 0.15)
        curve_key = jax.random.key(1)
        rows = []
        for ch in range(3):
            ck = jax.random.fold_in(curve_key, ch)
            noise = jax.random.normal(ck, (_N_CTRL - 2,), dtype=jnp.float32) * strengths[ch]
            ctrl_y = ctrl_x.at[1:-1].add(noise)
            ctrl_y = jnp.sort(ctrl_y)
            ctrl_y = jnp.clip(ctrl_y, -1.0, 1.0)
            ctrl_y = ctrl_y.at[0].set(-1.0).at[-1].set(1.0)
            slope = (ctrl_y[1:] - ctrl_y[:-1]) / (ctrl_x[1:] - ctrl_x[:-1] + 1e-8)
            rows.append(jnp.pad(slope * _THIRD, (0, 2)))
            rows.append(jnp.pad(ctrl_y[:-1], (0, 2)))
        tbl = jnp.broadcast_to(jnp.stack(rows)[:, :, None], (6, 8, w))
        tbl = jax.device_get(tbl)
    return jnp.asarray(tbl)


def _srgb_to_linear(c):
    u = (c + 0.055) * _INV_1_055
    return jnp.where(c <= 0.04045, c * _INV_12_92, jnp.exp(2.4 * jnp.log(u)))


def _cbrt_pos(x):
    return jnp.exp(jnp.log(x) * _THIRD)


def _apply_pwl(v, tbl_ref, ch):
    t = (v + 1.0) * 3.0
    tf = jnp.minimum(jnp.floor(t), float(_N_CTRL - 2))
    idx = tf.astype(jnp.int32)
    s = jnp.take_along_axis(tbl_ref[2 * ch], idx, axis=0)
    y = jnp.take_along_axis(tbl_ref[2 * ch + 1], idx, axis=0)
    return y + (t - tf) * s


def _flip_lanes(v):
    n = _WB
    h, w = v.shape
    rev = (n - 1) - jax.lax.broadcasted_iota(jnp.int32, (h, n), 1)
    tiles = []
    for t in range(w // n - 1, -1, -1):
        tiles.append(jnp.take_along_axis(v[:, t * n:(t + 1) * n], rev, axis=1))
    return jnp.concatenate(tiles, axis=-1)


def _body(tbl_ref, x_ref, out_in_ref, out_tg_ref, *, flip_w):
  for bi in range(x_ref.shape[0]):
    r = x_ref[bi, 0]
    g = x_ref[bi, 1]
    b = x_ref[bi, 2]
    if flip_w:
        r = _flip_lanes(r)
        g = _flip_lanes(g)
        b = _flip_lanes(b)

    rl = _srgb_to_linear(r)
    gl = _srgb_to_linear(g)
    bl = _srgb_to_linear(b)

    l = 0.4122214708 * rl + 0.5363325363 * gl + 0.0514459929 * bl
    m = 0.2119034982 * rl + 0.6806995451 * gl + 0.1073969566 * bl
    s = 0.0883024619 * rl + 0.2817188376 * gl + 0.6299787005 * bl
    l_ = _cbrt_pos(l)
    m_ = _cbrt_pos(m)
    s_ = _cbrt_pos(s)

    L2 = (2.0 * 0.2104542553) * l_ + (2.0 * 0.7936177850) * m_ - (2.0 * 0.0040720468) * s_
    a2 = (2.5 * 1.9779984951) * l_ - (2.5 * 2.4285922050) * m_ + (2.5 * 0.4505937099) * s_
    b2 = (2.5 * 0.0259040371) * l_ + (2.5 * 0.7827717662) * m_ - (2.5 * 0.8086757660) * s_

    Lp = jnp.clip(L2 - 1.0, -1.0, 1.0)
    ap = jnp.clip(a2, -1.0, 1.0)
    bp = jnp.clip(b2, -1.0, 1.0)

    out_tg_ref[bi, 0] = Lp
    out_tg_ref[bi, 1] = ap
    out_tg_ref[bi, 2] = bp

    out_in_ref[bi, 0] = _apply_pwl(Lp, tbl_ref, 0)
    out_in_ref[bi, 1] = _apply_pwl(ap, tbl_ref, 1)
    out_in_ref[bi, 2] = _apply_pwl(bp, tbl_ref, 2)


@jax.jit
def kernel(x):
    B, C, H, W = x.shape

    rng = np.random.RandomState(42)
    flips = rng.rand(3) < 0.5
    if flips[1]:
        x = jnp.flip(x, axis=2)
    if flips[2]:
        x = jnp.rot90(x, 1, axes=(2, 3))
        B, C, H, W = x.shape

    tables = _curve_tables(W)

    hb = 512
    grid = (B, H // hb)
    spec = pl.BlockSpec((1, C, hb, W), lambda i, j: (i, 0, j, 0))
    out = pl.pallas_call(
        functools.partial(_body, flip_w=bool(flips[0])),
        grid=grid,
        in_specs=[
            pl.BlockSpec((6, 8, W), lambda i, j: (0, 0, 0)),
            spec,
        ],
        out_specs=[spec, spec],
        out_shape=[
            jax.ShapeDtypeStruct((B, C, H, W), x.dtype),
            jax.ShapeDtypeStruct((B, C, H, W), x.dtype),
        ],
        compiler_params=pltpu.CompilerParams(
            dimension_semantics=("parallel", "parallel"),
        ),
    )(tables, x)
    return (out[0], out[1])


# speedup vs baseline: 1.2412x; 1.0207x over previous
import functools

import numpy as np
import jax
import jax.numpy as jnp
from jax.experimental import pallas as pl
from jax.experimental.pallas import tpu as pltpu

_N_CTRL = 7
_CTRL_X = np.linspace(-1.0, 1.0, _N_CTRL).astype(np.float32)
_DX = (_CTRL_X[1:] - _CTRL_X[:-1]).astype(np.float32)

_INV_12_92 = 1.0 / 12.92
_INV_1_055 = 1.0 / 1.055
_THIRD = 1.0 / 3.0

_WB = 128


def _curve_tables(w):
    with jax.ensure_compile_time_eval():
        ctrl_x = jnp.linspace(-1.0, 1.0, _N_CTRL)
        strengths = (0.25, 0.15, 0.15)
        curve_key = jax.random.key(1)
        rows = []
        for ch in range(3):
            ck = jax.random.fold_in(curve_key, ch)
            noise = jax.random.normal(ck, (_N_CTRL - 2,), dtype=jnp.float32) * strengths[ch]
            ctrl_y = ctrl_x.at[1:-1].add(noise)
            ctrl_y = jnp.sort(ctrl_y)
            ctrl_y = jnp.clip(ctrl_y, -1.0, 1.0)
            ctrl_y = ctrl_y.at[0].set(-1.0).at[-1].set(1.0)
            slope = (ctrl_y[1:] - ctrl_y[:-1]) / (ctrl_x[1:] - ctrl_x[:-1] + 1e-8)
            rows.append(jnp.pad(slope * _THIRD, (0, 2)))
            rows.append(jnp.pad(ctrl_y[:-1], (0, 2)).at[6].set(1.0))
        tbl = jnp.broadcast_to(jnp.stack(rows)[:, :, None], (6, 8, w))
        tbl = jax.device_get(tbl)
    return jnp.asarray(tbl)


def _srgb_to_linear(c):
    u = (c + 0.055) * _INV_1_055
    return jnp.where(c <= 0.04045, c * _INV_12_92, jnp.exp(2.4 * jnp.log(u)))


def _cbrt_pos(x):
    return jnp.exp(jnp.log(x) * _THIRD)


def _apply_pwl(v, tbl_ref, ch):
    t = (v + 1.0) * 3.0
    tf = jnp.floor(t)
    idx = tf.astype(jnp.int32)
    s = jnp.take_along_axis(tbl_ref[2 * ch], idx, axis=0)
    y = jnp.take_along_axis(tbl_ref[2 * ch + 1], idx, axis=0)
    return y + (t - tf) * s


def _flip_lanes(v):
    n = _WB
    h, w = v.shape
    rev = (n - 1) - jax.lax.broadcasted_iota(jnp.int32, (h, n), 1)
    tiles = []
    for t in range(w // n - 1, -1, -1):
        tiles.append(jnp.take_along_axis(v[:, t * n:(t + 1) * n], rev, axis=1))
    return jnp.concatenate(tiles, axis=-1)


def _body(tbl_ref, x_ref, out_in_ref, out_tg_ref, *, flip_w):
  for bi in range(x_ref.shape[0]):
    r = x_ref[bi, 0]
    g = x_ref[bi, 1]
    b = x_ref[bi, 2]
    if flip_w:
        r = _flip_lanes(r)
        g = _flip_lanes(g)
        b = _flip_lanes(b)

    rl = _srgb_to_linear(r)
    gl = _srgb_to_linear(g)
    bl = _srgb_to_linear(b)

    l = 0.4122214708 * rl + 0.5363325363 * gl + 0.0514459929 * bl
    m = 0.2119034982 * rl + 0.6806995451 * gl + 0.1073969566 * bl
    s = 0.0883024619 * rl + 0.2817188376 * gl + 0.6299787005 * bl
    l_ = _cbrt_pos(l)
    m_ = _cbrt_pos(m)
    s_ = _cbrt_pos(s)

    L2 = (2.0 * 0.2104542553) * l_ + (2.0 * 0.7936177850) * m_ - (2.0 * 0.0040720468) * s_
    a2 = (2.5 * 1.9779984951) * l_ - (2.5 * 2.4285922050) * m_ + (2.5 * 0.4505937099) * s_
    b2 = (2.5 * 0.0259040371) * l_ + (2.5 * 0.7827717662) * m_ - (2.5 * 0.8086757660) * s_

    Lp = jnp.clip(L2 - 1.0, -1.0, 1.0)
    ap = jnp.clip(a2, -1.0, 1.0)
    bp = jnp.clip(b2, -1.0, 1.0)

    out_tg_ref[bi, 0] = Lp
    out_tg_ref[bi, 1] = ap
    out_tg_ref[bi, 2] = bp

    out_in_ref[bi, 0] = _apply_pwl(Lp, tbl_ref, 0)
    out_in_ref[bi, 1] = _apply_pwl(ap, tbl_ref, 1)
    out_in_ref[bi, 2] = _apply_pwl(bp, tbl_ref, 2)


@jax.jit
def kernel(x):
    B, C, H, W = x.shape

    rng = np.random.RandomState(42)
    flips = rng.rand(3) < 0.5
    if flips[1]:
        x = jnp.flip(x, axis=2)
    if flips[2]:
        x = jnp.rot90(x, 1, axes=(2, 3))
        B, C, H, W = x.shape

    tables = _curve_tables(W)

    hb = 512
    grid = (B, H // hb)
    spec = pl.BlockSpec((1, C, hb, W), lambda i, j: (i, 0, j, 0))
    out = pl.pallas_call(
        functools.partial(_body, flip_w=bool(flips[0])),
        grid=grid,
        in_specs=[
            pl.BlockSpec((6, 8, W), lambda i, j: (0, 0, 0)),
            spec,
        ],
        out_specs=[spec, spec],
        out_shape=[
            jax.ShapeDtypeStruct((B, C, H, W), x.dtype),
            jax.ShapeDtypeStruct((B, C, H, W), x.dtype),
        ],
        compiler_params=pltpu.CompilerParams(
            dimension_semantics=("parallel", "parallel"),
        ),
    )(tables, x)
    return (out[0], out[1])
